# baseline (device time: 129210 ns/iter reference)
import jax
import jax.numpy as jnp
from jax import lax
from jax.experimental import pallas as pl
from jax.experimental.pallas import tpu as pltpu

N_DEV = 4
N_EXP = 16
EXP_PER_DEV = 4
CAPACITY = 204
CAP_G = 384


def kernel(x, router_W, route_idx, expert_W):
    del router_W
    n_tok, d_model = x.shape
    _, _, d_ff = expert_W.shape

    xb = x.astype(jnp.bfloat16)
    wb = expert_W.astype(jnp.bfloat16)
    route_f = route_idx.astype(jnp.float32)

    oh = (route_idx == jnp.arange(N_EXP, dtype=jnp.int32)[None, :]).astype(
        jnp.float32
    )
    rank = jnp.take_along_axis(jnp.cumsum(oh, axis=0) - oh, route_idx, axis=1)
    cnts = jnp.zeros((1, 128), jnp.float32).at[0, :N_EXP].set(oh.sum(axis=0))

    group = route_idx // EXP_PER_DEV
    oh_g = (group == jnp.arange(N_DEV, dtype=jnp.int32)[None, :]).astype(
        jnp.float32
    )
    rank_g = jnp.take_along_axis(
        jnp.cumsum(oh_g, axis=0) - oh_g, group, axis=1
    ).astype(jnp.int32)
    slot = jnp.where(
        rank_g < CAP_G, group * CAP_G + rank_g, N_DEV * CAP_G
    )[:, 0]

    Xc = (
        jnp.zeros((N_DEV * CAP_G, d_model), jnp.bfloat16)
        .at[slot].set(xb, mode="drop")
        .reshape(N_DEV, CAP_G, d_model)
    )
    exp_slot = (
        jnp.zeros((N_DEV * CAP_G, 1), jnp.float32)
        .at[slot].set(route_f, mode="drop")
        .reshape(N_DEV, CAP_G, 1)
    )
    rank_slot = (
        jnp.zeros((N_DEV * CAP_G, 1), jnp.float32)
        .at[slot].set(rank, mode="drop")
        .reshape(N_DEV, CAP_G, 1)
    )
    ohs = (
        exp_slot == jnp.arange(N_EXP, dtype=jnp.float32)[None, None, :]
    ).astype(jnp.float32)

    def body(
        xc_ref, w_ref, exp_ref, rank_ref, ohs_ref, cnt_ref, y_ref,
        comm_ref, send_sems, recv_sems, cnt_send_sems, cnt_recv_sems,
        counts_all,
    ):
        my = lax.axis_index("i")
        right = jnp.mod(my + 1, N_DEV)
        left = jnp.mod(my - 1, N_DEV)

        barrier = pltpu.get_barrier_semaphore()
        for o in range(1, N_DEV):
            pl.semaphore_signal(
                barrier, inc=1,
                device_id=(jnp.mod(my + o, N_DEV),),
                device_id_type=pl.DeviceIdType.MESH,
            )
        pl.semaphore_wait(barrier, N_DEV - 1)

        cnt_sends = []
        for o in range(1, N_DEV):
            tgt = jnp.mod(my + o, N_DEV)
            s = pltpu.make_async_remote_copy(
                src_ref=cnt_ref,
                dst_ref=counts_all.at[pl.ds(my, 1)],
                send_sem=cnt_send_sems.at[o],
                recv_sem=cnt_recv_sems.at[my],
                device_id=(tgt,),
                device_id_type=pl.DeviceIdType.MESH,
            )
            s.start()
            cnt_sends.append(s)

        r1f = pltpu.make_async_remote_copy(
            src_ref=w_ref, dst_ref=comm_ref.at[0],
            send_sem=send_sems.at[0], recv_sem=recv_sems.at[0],
            device_id=(right,), device_id_type=pl.DeviceIdType.MESH,
        )
        r1r = pltpu.make_async_remote_copy(
            src_ref=w_ref, dst_ref=comm_ref.at[1],
            send_sem=send_sems.at[1], recv_sem=recv_sems.at[1],
            device_id=(left,), device_id_type=pl.DeviceIdType.MESH,
        )
        r1f.start()
        r1r.start()

        for o in range(1, N_DEV):
            src_dev = jnp.mod(my + o, N_DEV)
            r = pltpu.make_async_remote_copy(
                src_ref=cnt_ref,
                dst_ref=counts_all.at[pl.ds(src_dev, 1)],
                send_sem=cnt_send_sems.at[0],
                recv_sem=cnt_recv_sems.at[src_dev],
                device_id=(src_dev,),
                device_id_type=pl.DeviceIdType.MESH,
            )
            r.wait_recv()
        for s in cnt_sends:
            s.wait_send()

        ca = counts_all[...]
        dev_iota = lax.broadcasted_iota(jnp.int32, (N_DEV, 128), 0)
        prior = jnp.where(dev_iota < my, ca, 0.0).sum(
            axis=0, keepdims=True
        )[:, :N_EXP]

        def do_hop(chunk_ref, src_dev):
            xv = xc_ref[pl.ds(src_dev, 1)].reshape(CAP_G, d_model)
            ev = exp_ref[pl.ds(src_dev, 1)].reshape(CAP_G, 1)
            acc = None
            for k in range(EXP_PER_DEV):
                e = (src_dev * EXP_PER_DEV + k).astype(jnp.float32)
                xm = xv * (ev == e).astype(jnp.bfloat16)
                d = jnp.dot(xm, chunk_ref[k], preferred_element_type=jnp.float32)
                acc = d if acc is None else acc + d
            offs = (
                ohs_ref[pl.ds(src_dev, 1)].reshape(CAP_G, N_EXP) * prior
            ).sum(axis=-1, keepdims=True)
            rk = rank_ref[pl.ds(src_dev, 1)].reshape(CAP_G, 1)
            kslice = ((offs + rk) < float(CAPACITY)).astype(jnp.float32)
            y_ref[pl.ds(src_dev, 1)] = (acc * kslice)[None]

        do_hop(w_ref, my)

        r1f.wait()
        r1r.wait()

        half = EXP_PER_DEV // 2
        r2f = pltpu.make_async_remote_copy(
            src_ref=comm_ref.at[0, pl.ds(0, half)],
            dst_ref=comm_ref.at[2, pl.ds(0, half)],
            send_sem=send_sems.at[2], recv_sem=recv_sems.at[2],
            device_id=(right,), device_id_type=pl.DeviceIdType.MESH,
        )
        r2r = pltpu.make_async_remote_copy(
            src_ref=comm_ref.at[1, pl.ds(half, half)],
            dst_ref=comm_ref.at[2, pl.ds(half, half)],
            send_sem=send_sems.at[3], recv_sem=recv_sems.at[3],
            device_id=(left,), device_id_type=pl.DeviceIdType.MESH,
        )
        r2f.start()
        r2r.start()

        do_hop(comm_ref.at[0], left)
        do_hop(comm_ref.at[1], right)

        r2f.wait()
        r2r.wait()

        do_hop(comm_ref.at[2], jnp.mod(my + 2, N_DEV))

    y = pl.pallas_call(
        body,
        out_shape=jax.ShapeDtypeStruct((N_DEV, CAP_G, d_ff), jnp.float32),
        in_specs=[pl.BlockSpec(memory_space=pltpu.VMEM)] * 6,
        out_specs=pl.BlockSpec(memory_space=pltpu.VMEM),
        scratch_shapes=[
            pltpu.VMEM((3, EXP_PER_DEV, d_model, d_ff), jnp.bfloat16),
            pltpu.SemaphoreType.DMA((4,)),
            pltpu.SemaphoreType.DMA((4,)),
            pltpu.SemaphoreType.DMA((N_DEV,)),
            pltpu.SemaphoreType.DMA((N_DEV,)),
            pltpu.VMEM((N_DEV, 128), jnp.float32),
        ],
        compiler_params=pltpu.CompilerParams(collective_id=0),
    )(Xc, wb, exp_slot, rank_slot, ohs, cnts)

    return jnp.take(
        y.reshape(N_DEV * CAP_G, d_ff), slot, axis=0,
        mode="fill", fill_value=0.0,
    )


# device time: 85089 ns/iter; 1.5185x vs baseline; 1.5185x over previous
import jax
import jax.numpy as jnp
from jax import lax
from jax.experimental import pallas as pl
from jax.experimental.pallas import tpu as pltpu

N_DEV = 4
N_EXP = 16
EXP_PER_DEV = 4
CAPACITY = 204
CAP_G = 384
NSLOT = N_DEV * CAP_G
META_L = 128


def kernel(x, router_W, route_idx, expert_W):
    del router_W
    n_tok, d_model = x.shape
    _, _, d_ff = expert_W.shape

    xb = x.astype(jnp.bfloat16)
    wb = expert_W.astype(jnp.bfloat16)
    route_f = route_idx.astype(jnp.float32)

    oh = (route_idx == jnp.arange(N_EXP, dtype=jnp.int32)[None, :]).astype(
        jnp.float32
    )
    rank = jnp.take_along_axis(jnp.cumsum(oh, axis=0) - oh, route_idx, axis=1)
    cnts = jnp.zeros((1, 128), jnp.float32).at[0, :N_EXP].set(oh.sum(axis=0))

    group = route_idx // EXP_PER_DEV
    oh_g = (group == jnp.arange(N_DEV, dtype=jnp.int32)[None, :]).astype(
        jnp.float32
    )
    rank_g = jnp.take_along_axis(
        jnp.cumsum(oh_g, axis=0) - oh_g, group, axis=1
    ).astype(jnp.int32)
    slot = jnp.where(rank_g < CAP_G, group * CAP_G + rank_g, NSLOT)[:, 0]

    P = (jnp.arange(NSLOT, dtype=jnp.int32)[:, None] == slot[None, :]).astype(
        jnp.bfloat16
    )
    PT = P.T
    meta_cols = jnp.concatenate(
        [route_f, jnp.floor(rank / 256.0), jnp.mod(rank, 256.0)], axis=1
    )
    meta = jnp.zeros((NSLOT, META_L), jnp.float32).at[:, :3].set(
        P.astype(jnp.float32) @ meta_cols
    )
    meta_b = meta.astype(jnp.bfloat16).reshape(N_DEV, CAP_G, META_L)

    D_BLK = d_model + META_L

    def body(
        xb_ref, wb_ref, p_ref, pt_ref, meta_ref, cnt_ref, out_ref,
        xc_s, xin_s, yb_s, yin_s,
        cnt_send, cnt_recv, xa_send, xa_recv, ya_send, ya_recv,
        counts_all,
    ):
        my = lax.axis_index("i")

        barrier = pltpu.get_barrier_semaphore()
        for o in range(1, N_DEV):
            pl.semaphore_signal(
                barrier, inc=1,
                device_id=(jnp.mod(my + o, N_DEV),),
                device_id_type=pl.DeviceIdType.MESH,
            )
        pl.semaphore_wait(barrier, N_DEV - 1)

        cnt_sends = []
        for o in range(1, N_DEV):
            tgt = jnp.mod(my + o, N_DEV)
            s = pltpu.make_async_remote_copy(
                src_ref=cnt_ref,
                dst_ref=counts_all.at[pl.ds(my, 1)],
                send_sem=cnt_send.at[o],
                recv_sem=cnt_recv.at[my],
                device_id=(tgt,),
                device_id_type=pl.DeviceIdType.MESH,
            )
            s.start()
            cnt_sends.append(s)

        xc_s[:, :, :d_model] = (
            jnp.dot(p_ref[...], xb_ref[...], preferred_element_type=jnp.float32)
            .astype(jnp.bfloat16)
            .reshape(N_DEV, CAP_G, d_model)
        )
        xc_s[:, :, d_model:] = meta_ref[...]

        xin_s[pl.ds(my, 1)] = xc_s[pl.ds(my, 1)]
        x_sends = []
        for o in range(1, N_DEV):
            tgt = jnp.mod(my + o, N_DEV)
            sx = pltpu.make_async_remote_copy(
                src_ref=xc_s.at[pl.ds(tgt, 1)],
                dst_ref=xin_s.at[pl.ds(my, 1)],
                send_sem=xa_send.at[o],
                recv_sem=xa_recv.at[my],
                device_id=(tgt,),
                device_id_type=pl.DeviceIdType.MESH,
            )
            sx.start()
            x_sends.append(sx)

        counts_all[pl.ds(my, 1)] = cnt_ref[...]
        for o in range(1, N_DEV):
            src_dev = jnp.mod(my + o, N_DEV)
            r = pltpu.make_async_remote_copy(
                src_ref=cnt_ref,
                dst_ref=counts_all.at[pl.ds(src_dev, 1)],
                send_sem=cnt_send.at[0],
                recv_sem=cnt_recv.at[src_dev],
                device_id=(src_dev,),
                device_id_type=pl.DeviceIdType.MESH,
            )
            r.wait_recv()

        ca = counts_all[...]
        dev_iota = lax.broadcasted_iota(jnp.int32, (N_DEV, 128), 0)
        lane16 = jnp.arange(N_EXP, dtype=jnp.int32)[None, :].astype(
            jnp.float32
        )

        def compute_block(s):
            blk = xin_s[pl.ds(s, 1)].reshape(CAP_G, D_BLK)
            xv = blk[:, :d_model]
            ev = blk[:, d_model:d_model + 1].astype(jnp.float32)
            rk = (
                blk[:, d_model + 1:d_model + 2].astype(jnp.float32) * 256.0
                + blk[:, d_model + 2:d_model + 3].astype(jnp.float32)
            )
            offs_vec = jnp.where(dev_iota < s, ca, 0.0).sum(
                axis=0, keepdims=True
            )[:, :N_EXP]
            ohs = (ev == lane16).astype(jnp.float32)
            offs_row = (ohs * offs_vec).sum(axis=1, keepdims=True)
            keep = (rk + offs_row) < float(CAPACITY)
            acc = None
            for k in range(EXP_PER_DEV):
                e_f = (my * EXP_PER_DEV + k).astype(jnp.float32)
                m = (ev == e_f) & keep
                xm = xv * m.astype(jnp.bfloat16)
                d = jnp.dot(
                    xm, wb_ref[k], preferred_element_type=jnp.float32
                )
                acc = d if acc is None else acc + d
            yb_s[pl.ds(s, 1)] = acc.astype(jnp.bfloat16)[None]

        compute_block(my)
        yin_s[pl.ds(my, 1)] = yb_s[pl.ds(my, 1)]

        y_sends = []
        for o in (1, 3, 2):
            s = jnp.mod(my + o, N_DEV)
            rx = pltpu.make_async_remote_copy(
                src_ref=xc_s.at[pl.ds(s, 1)],
                dst_ref=xin_s.at[pl.ds(s, 1)],
                send_sem=xa_send.at[0],
                recv_sem=xa_recv.at[s],
                device_id=(s,),
                device_id_type=pl.DeviceIdType.MESH,
            )
            rx.wait_recv()
            compute_block(s)
            sy = pltpu.make_async_remote_copy(
                src_ref=yb_s.at[pl.ds(s, 1)],
                dst_ref=yin_s.at[pl.ds(my, 1)],
                send_sem=ya_send.at[o],
                recv_sem=ya_recv.at[my],
                device_id=(s,),
                device_id_type=pl.DeviceIdType.MESH,
            )
            sy.start()
            y_sends.append(sy)

        for o in (1, 3, 2):
            g = jnp.mod(my + o, N_DEV)
            ry = pltpu.make_async_remote_copy(
                src_ref=yb_s.at[pl.ds(g, 1)],
                dst_ref=yin_s.at[pl.ds(g, 1)],
                send_sem=ya_send.at[0],
                recv_sem=ya_recv.at[g],
                device_id=(g,),
                device_id_type=pl.DeviceIdType.MESH,
            )
            ry.wait_recv()

        yf = yin_s[...].reshape(NSLOT, d_ff)
        out_ref[...] = jnp.dot(
            pt_ref[...], yf, preferred_element_type=jnp.float32
        )

        for s in cnt_sends + x_sends + y_sends:
            s.wait_send()

    return pl.pallas_call(
        body,
        out_shape=jax.ShapeDtypeStruct((n_tok, d_ff), jnp.float32),
        in_specs=[pl.BlockSpec(memory_space=pltpu.VMEM)] * 6,
        out_specs=pl.BlockSpec(memory_space=pltpu.VMEM),
        scratch_shapes=[
            pltpu.VMEM((N_DEV, CAP_G, D_BLK), jnp.bfloat16),
            pltpu.VMEM((N_DEV, CAP_G, D_BLK), jnp.bfloat16),
            pltpu.VMEM((N_DEV, CAP_G, d_ff), jnp.bfloat16),
            pltpu.VMEM((N_DEV, CAP_G, d_ff), jnp.bfloat16),
            pltpu.SemaphoreType.DMA((N_DEV,)),
            pltpu.SemaphoreType.DMA((N_DEV,)),
            pltpu.SemaphoreType.DMA((N_DEV,)),
            pltpu.SemaphoreType.DMA((N_DEV,)),
            pltpu.SemaphoreType.DMA((N_DEV,)),
            pltpu.SemaphoreType.DMA((N_DEV,)),
            pltpu.VMEM((N_DEV, 128), jnp.float32),
        ],
        compiler_params=pltpu.CompilerParams(collective_id=0),
    )(xb, wb, P, PT, meta_b, cnts)


# device time: 74055 ns/iter; 1.7448x vs baseline; 1.1490x over previous
import jax
import jax.numpy as jnp
from jax import lax
from jax.experimental import pallas as pl
from jax.experimental.pallas import tpu as pltpu

N_DEV = 4
N_EXP = 16
EXP_PER_DEV = 4
CAPACITY = 204
CAP_G = 384
NSLOT = N_DEV * CAP_G
META_L = 128


def kernel(x, router_W, route_idx, expert_W):
    del router_W
    n_tok, d_model = x.shape
    _, _, d_ff = expert_W.shape

    xb = x.astype(jnp.bfloat16)
    wb = expert_W.astype(jnp.bfloat16)
    route_f = route_idx.astype(jnp.float32)

    oh = (route_idx == jnp.arange(N_EXP, dtype=jnp.int32)[None, :]).astype(
        jnp.float32
    )
    rank = ((jnp.cumsum(oh, axis=0) - oh) * oh).sum(axis=1, keepdims=True)
    cnts = jnp.zeros((1, 128), jnp.float32).at[0, :N_EXP].set(oh.sum(axis=0))

    group = route_idx // EXP_PER_DEV
    oh_g = (group == jnp.arange(N_DEV, dtype=jnp.int32)[None, :]).astype(
        jnp.float32
    )
    rank_g = ((jnp.cumsum(oh_g, axis=0) - oh_g) * oh_g).sum(
        axis=1, keepdims=True
    ).astype(jnp.int32)
    slot = jnp.where(
        rank_g < CAP_G, group * CAP_G + rank_g, NSLOT
    )
    slot_row = slot.reshape(1, n_tok)

    meta_b = (
        jnp.zeros((n_tok, META_L), jnp.float32)
        .at[:, 0:1].set(route_f)
        .at[:, 1:2].set(jnp.floor(rank / 256.0))
        .at[:, 2:3].set(jnp.mod(rank, 256.0))
    ).astype(jnp.bfloat16)

    D_BLK = d_model + META_L

    def body(
        xb_ref, wb_ref, slot_row_ref, slot_col_ref, meta_ref, cnt_ref,
        out_ref,
        xc_s, xin_s, yb_s, yin_s,
        cnt_send, cnt_recv, xa_send, xa_recv, ya_send, ya_recv,
        counts_all,
    ):
        my = lax.axis_index("i")

        barrier = pltpu.get_barrier_semaphore()
        for o in range(1, N_DEV):
            pl.semaphore_signal(
                barrier, inc=1,
                device_id=(jnp.mod(my + o, N_DEV),),
                device_id_type=pl.DeviceIdType.MESH,
            )
        pl.semaphore_wait(barrier, N_DEV - 1)

        cnt_sends = []
        for o in range(1, N_DEV):
            tgt = jnp.mod(my + o, N_DEV)
            s = pltpu.make_async_remote_copy(
                src_ref=cnt_ref,
                dst_ref=counts_all.at[pl.ds(my, 1)],
                send_sem=cnt_send.at[o],
                recv_sem=cnt_recv.at[my],
                device_id=(tgt,),
                device_id_type=pl.DeviceIdType.MESH,
            )
            s.start()
            cnt_sends.append(s)

        pmat = (
            lax.broadcasted_iota(jnp.int32, (NSLOT, n_tok), 0)
            == slot_row_ref[...]
        ).astype(jnp.bfloat16)
        xc_s[:, :, :d_model] = (
            jnp.dot(pmat, xb_ref[...], preferred_element_type=jnp.float32)
            .astype(jnp.bfloat16)
            .reshape(N_DEV, CAP_G, d_model)
        )
        xc_s[:, :, d_model:] = (
            jnp.dot(pmat, meta_ref[...], preferred_element_type=jnp.float32)
            .astype(jnp.bfloat16)
            .reshape(N_DEV, CAP_G, META_L)
        )

        xin_s[pl.ds(my, 1)] = xc_s[pl.ds(my, 1)]
        x_sends = []
        for o in range(1, N_DEV):
            tgt = jnp.mod(my + o, N_DEV)
            sx = pltpu.make_async_remote_copy(
                src_ref=xc_s.at[pl.ds(tgt, 1)],
                dst_ref=xin_s.at[pl.ds(my, 1)],
                send_sem=xa_send.at[o],
                recv_sem=xa_recv.at[my],
                device_id=(tgt,),
                device_id_type=pl.DeviceIdType.MESH,
            )
            sx.start()
            x_sends.append(sx)

        counts_all[pl.ds(my, 1)] = cnt_ref[...]
        for o in range(1, N_DEV):
            src_dev = jnp.mod(my + o, N_DEV)
            r = pltpu.make_async_remote_copy(
                src_ref=cnt_ref,
                dst_ref=counts_all.at[pl.ds(src_dev, 1)],
                send_sem=cnt_send.at[0],
                recv_sem=cnt_recv.at[src_dev],
                device_id=(src_dev,),
                device_id_type=pl.DeviceIdType.MESH,
            )
            r.wait_recv()

        ca = counts_all[...]
        dev_iota = lax.broadcasted_iota(jnp.int32, (N_DEV, 128), 0)
        lane16 = jnp.arange(N_EXP, dtype=jnp.int32)[None, :].astype(
            jnp.float32
        )

        def compute_block(s):
            blk = xin_s[pl.ds(s, 1)].reshape(CAP_G, D_BLK)
            xv = blk[:, :d_model]
            ev = blk[:, d_model:d_model + 1].astype(jnp.float32)
            rk = (
                blk[:, d_model + 1:d_model + 2].astype(jnp.float32) * 256.0
                + blk[:, d_model + 2:d_model + 3].astype(jnp.float32)
            )
            offs_vec = jnp.where(dev_iota < s, ca, 0.0).sum(
                axis=0, keepdims=True
            )[:, :N_EXP]
            ohs = (ev == lane16).astype(jnp.float32)
            offs_row = (ohs * offs_vec).sum(axis=1, keepdims=True)
            keep = (rk + offs_row) < float(CAPACITY)
            acc = None
            for k in range(EXP_PER_DEV):
                e_f = (my * EXP_PER_DEV + k).astype(jnp.float32)
                m = (ev == e_f) & keep
                xm = xv * m.astype(jnp.bfloat16)
                d = jnp.dot(
                    xm, wb_ref[k], preferred_element_type=jnp.float32
                )
                acc = d if acc is None else acc + d
            yb_s[pl.ds(s, 1)] = acc.astype(jnp.bfloat16)[None]

        compute_block(my)
        yin_s[pl.ds(my, 1)] = yb_s[pl.ds(my, 1)]

        y_sends = []
        for o in (1, 3, 2):
            s = jnp.mod(my + o, N_DEV)
            rx = pltpu.make_async_remote_copy(
                src_ref=xc_s.at[pl.ds(s, 1)],
                dst_ref=xin_s.at[pl.ds(s, 1)],
                send_sem=xa_send.at[0],
                recv_sem=xa_recv.at[s],
                device_id=(s,),
                device_id_type=pl.DeviceIdType.MESH,
            )
            rx.wait_recv()
            compute_block(s)
            sy = pltpu.make_async_remote_copy(
                src_ref=yb_s.at[pl.ds(s, 1)],
                dst_ref=yin_s.at[pl.ds(my, 1)],
                send_sem=ya_send.at[o],
                recv_sem=ya_recv.at[my],
                device_id=(s,),
                device_id_type=pl.DeviceIdType.MESH,
            )
            sy.start()
            y_sends.append(sy)

        for o in (1, 3, 2):
            g = jnp.mod(my + o, N_DEV)
            ry = pltpu.make_async_remote_copy(
                src_ref=yb_s.at[pl.ds(g, 1)],
                dst_ref=yin_s.at[pl.ds(g, 1)],
                send_sem=ya_send.at[0],
                recv_sem=ya_recv.at[g],
                device_id=(g,),
                device_id_type=pl.DeviceIdType.MESH,
            )
            ry.wait_recv()

        ptmat = (
            slot_col_ref[...]
            == lax.broadcasted_iota(jnp.int32, (n_tok, NSLOT), 1)
        ).astype(jnp.bfloat16)
        yf = yin_s[...].reshape(NSLOT, d_ff)
        out_ref[...] = jnp.dot(
            ptmat, yf, preferred_element_type=jnp.float32
        )

        for s in cnt_sends + x_sends + y_sends:
            s.wait_send()

    return pl.pallas_call(
        body,
        out_shape=jax.ShapeDtypeStruct((n_tok, d_ff), jnp.float32),
        in_specs=[pl.BlockSpec(memory_space=pltpu.VMEM)] * 6,
        out_specs=pl.BlockSpec(memory_space=pltpu.VMEM),
        scratch_shapes=[
            pltpu.VMEM((N_DEV, CAP_G, D_BLK), jnp.bfloat16),
            pltpu.VMEM((N_DEV, CAP_G, D_BLK), jnp.bfloat16),
            pltpu.VMEM((N_DEV, CAP_G, d_ff), jnp.bfloat16),
            pltpu.VMEM((N_DEV, CAP_G, d_ff), jnp.bfloat16),
            pltpu.SemaphoreType.DMA((N_DEV,)),
            pltpu.SemaphoreType.DMA((N_DEV,)),
            pltpu.SemaphoreType.DMA((N_DEV,)),
            pltpu.SemaphoreType.DMA((N_DEV,)),
            pltpu.SemaphoreType.DMA((N_DEV,)),
            pltpu.SemaphoreType.DMA((N_DEV,)),
            pltpu.VMEM((N_DEV, 128), jnp.float32),
        ],
        compiler_params=pltpu.CompilerParams(collective_id=0),
    )(xb, wb, slot_row, slot, meta_b, cnts)


# device time: 60166 ns/iter; 2.1476x vs baseline; 1.2308x over previous
import jax
import jax.numpy as jnp
from jax import lax
from jax.experimental import pallas as pl
from jax.experimental.pallas import tpu as pltpu

N_DEV = 4
N_EXP = 16
EXP_PER_DEV = 4
CAPACITY = 204
CAP_G = 384
NSLOT = N_DEV * CAP_G
META_L = 128


def kernel(x, router_W, route_idx, expert_W):
    del router_W
    n_tok, d_model = x.shape
    _, _, d_ff = expert_W.shape

    xb = x.astype(jnp.bfloat16)
    wb = expert_W.astype(jnp.bfloat16)
    route_col = route_idx
    route_row = route_idx.reshape(1, n_tok)

    D_BLK = d_model + META_L

    def body(
        xb_ref, wb_ref, rc_ref, rr_ref, out_ref,
        xc_s, xin_s, yb_s, yin_s, cnt_s,
        cnt_send, cnt_recv, xa_send, xa_recv, ya_send, ya_recv,
        counts_all,
    ):
        my = lax.axis_index("i")

        barrier = pltpu.get_barrier_semaphore()
        for o in range(1, N_DEV):
            pl.semaphore_signal(
                barrier, inc=1,
                device_id=(jnp.mod(my + o, N_DEV),),
                device_id_type=pl.DeviceIdType.MESH,
            )
        pl.semaphore_wait(barrier, N_DEV - 1)

        rc = rc_ref[...]
        rr = rr_ref[...]

        oh128 = (
            rc == lax.broadcasted_iota(jnp.int32, (n_tok, 128), 1)
        ).astype(jnp.float32)
        cnt_s[...] = oh128.sum(axis=0, keepdims=True)

        cnt_sends = []
        for o in range(1, N_DEV):
            tgt = jnp.mod(my + o, N_DEV)
            s = pltpu.make_async_remote_copy(
                src_ref=cnt_s,
                dst_ref=counts_all.at[pl.ds(my, 1)],
                send_sem=cnt_send.at[o],
                recv_sem=cnt_recv.at[my],
                device_id=(tgt,),
                device_id_type=pl.DeviceIdType.MESH,
            )
            s.start()
            cnt_sends.append(s)

        lower = (
            lax.broadcasted_iota(jnp.int32, (n_tok, n_tok), 0)
            >= lax.broadcasted_iota(jnp.int32, (n_tok, n_tok), 1)
        ).astype(jnp.bfloat16)

        oh16 = (
            rc == lax.broadcasted_iota(jnp.int32, (n_tok, N_EXP), 1)
        )
        oh16b = oh16.astype(jnp.bfloat16)
        oh16f = oh16.astype(jnp.float32)
        csum16 = jnp.dot(lower, oh16b, preferred_element_type=jnp.float32)
        rank = ((csum16 - oh16f) * oh16f).sum(axis=1, keepdims=True)

        gc = rc // EXP_PER_DEV
        ohg_c = (
            gc == lax.broadcasted_iota(jnp.int32, (n_tok, N_DEV), 1)
        )
        ohg_cb = ohg_c.astype(jnp.bfloat16)
        ohg_cf = ohg_c.astype(jnp.float32)
        csg_c = jnp.dot(lower, ohg_cb, preferred_element_type=jnp.float32)
        rkg_c = ((csg_c - ohg_cf) * ohg_cf).sum(axis=1, keepdims=True)
        slot_col = jnp.where(
            rkg_c < CAP_G,
            gc.astype(jnp.float32) * CAP_G + rkg_c,
            float(NSLOT),
        )

        upper = (
            lax.broadcasted_iota(jnp.int32, (n_tok, n_tok), 0)
            <= lax.broadcasted_iota(jnp.int32, (n_tok, n_tok), 1)
        ).astype(jnp.bfloat16)
        gr = rr // EXP_PER_DEV
        ohg_r = (
            gr == lax.broadcasted_iota(jnp.int32, (N_DEV, n_tok), 0)
        )
        ohg_rb = ohg_r.astype(jnp.bfloat16)
        ohg_rf = ohg_r.astype(jnp.float32)
        csg_r = jnp.dot(ohg_rb, upper, preferred_element_type=jnp.float32)
        rkg_r = ((csg_r - ohg_rf) * ohg_rf).sum(axis=0, keepdims=True)
        slot_row = jnp.where(
            rkg_r < CAP_G,
            gr.astype(jnp.float32) * CAP_G + rkg_r,
            float(NSLOT),
        )

        rhi = jnp.floor(rank / 256.0)
        rlo = rank - 256.0 * rhi
        lane = lax.broadcasted_iota(jnp.int32, (1, META_L), 1)
        meta = (
            rc.astype(jnp.float32) * (lane == 0)
            + rhi * (lane == 1)
            + rlo * (lane == 2)
        ).astype(jnp.bfloat16)

        pmat = (
            lax.broadcasted_iota(jnp.int32, (NSLOT, n_tok), 0).astype(
                jnp.float32
            )
            == slot_row
        ).astype(jnp.bfloat16)
        xc_s[:, :, :d_model] = (
            jnp.dot(pmat, xb_ref[...], preferred_element_type=jnp.float32)
            .astype(jnp.bfloat16)
            .reshape(N_DEV, CAP_G, d_model)
        )
        xc_s[:, :, d_model:] = (
            jnp.dot(pmat, meta, preferred_element_type=jnp.float32)
            .astype(jnp.bfloat16)
            .reshape(N_DEV, CAP_G, META_L)
        )

        xin_s[pl.ds(my, 1)] = xc_s[pl.ds(my, 1)]
        x_sends = []
        for o in range(1, N_DEV):
            tgt = jnp.mod(my + o, N_DEV)
            sx = pltpu.make_async_remote_copy(
                src_ref=xc_s.at[pl.ds(tgt, 1)],
                dst_ref=xin_s.at[pl.ds(my, 1)],
                send_sem=xa_send.at[o],
                recv_sem=xa_recv.at[my],
                device_id=(tgt,),
                device_id_type=pl.DeviceIdType.MESH,
            )
            sx.start()
            x_sends.append(sx)

        counts_all[pl.ds(my, 1)] = cnt_s[...]
        for o in range(1, N_DEV):
            src_dev = jnp.mod(my + o, N_DEV)
            r = pltpu.make_async_remote_copy(
                src_ref=cnt_s,
                dst_ref=counts_all.at[pl.ds(src_dev, 1)],
                send_sem=cnt_send.at[0],
                recv_sem=cnt_recv.at[src_dev],
                device_id=(src_dev,),
                device_id_type=pl.DeviceIdType.MESH,
            )
            r.wait_recv()

        ca = counts_all[...]
        dev_iota = lax.broadcasted_iota(jnp.int32, (N_DEV, 128), 0)
        lane16 = lax.broadcasted_iota(jnp.int32, (1, N_EXP), 1).astype(
            jnp.float32
        )

        def compute_block(s):
            blk = xin_s[pl.ds(s, 1)].reshape(CAP_G, D_BLK)
            xv = blk[:, :d_model]
            ev = blk[:, d_model:d_model + 1].astype(jnp.float32)
            rk = (
                blk[:, d_model + 1:d_model + 2].astype(jnp.float32) * 256.0
                + blk[:, d_model + 2:d_model + 3].astype(jnp.float32)
            )
            offs_vec = jnp.where(dev_iota < s, ca, 0.0).sum(
                axis=0, keepdims=True
            )[:, :N_EXP]
            ohs = (ev == lane16).astype(jnp.float32)
            offs_row = (ohs * offs_vec).sum(axis=1, keepdims=True)
            keep = (rk + offs_row) < float(CAPACITY)
            acc = None
            for k in range(EXP_PER_DEV):
                e_f = (my * EXP_PER_DEV + k).astype(jnp.float32)
                m = (ev == e_f) & keep
                xm = xv * m.astype(jnp.bfloat16)
                d = jnp.dot(
                    xm, wb_ref[k], preferred_element_type=jnp.float32
                )
                acc = d if acc is None else acc + d
            yb_s[pl.ds(s, 1)] = acc.astype(jnp.bfloat16)[None]

        compute_block(my)
        yin_s[pl.ds(my, 1)] = yb_s[pl.ds(my, 1)]

        y_sends = []
        for o in (1, 3, 2):
            s = jnp.mod(my + o, N_DEV)
            rx = pltpu.make_async_remote_copy(
                src_ref=xc_s.at[pl.ds(s, 1)],
                dst_ref=xin_s.at[pl.ds(s, 1)],
                send_sem=xa_send.at[0],
                recv_sem=xa_recv.at[s],
                device_id=(s,),
                device_id_type=pl.DeviceIdType.MESH,
            )
            rx.wait_recv()
            compute_block(s)
            sy = pltpu.make_async_remote_copy(
                src_ref=yb_s.at[pl.ds(s, 1)],
                dst_ref=yin_s.at[pl.ds(my, 1)],
                send_sem=ya_send.at[o],
                recv_sem=ya_recv.at[my],
                device_id=(s,),
                device_id_type=pl.DeviceIdType.MESH,
            )
            sy.start()
            y_sends.append(sy)

        for o in (1, 3, 2):
            g = jnp.mod(my + o, N_DEV)
            ry = pltpu.make_async_remote_copy(
                src_ref=yb_s.at[pl.ds(g, 1)],
                dst_ref=yin_s.at[pl.ds(g, 1)],
                send_sem=ya_send.at[0],
                recv_sem=ya_recv.at[g],
                device_id=(g,),
                device_id_type=pl.DeviceIdType.MESH,
            )
            ry.wait_recv()

        ptmat = (
            slot_col
            == lax.broadcasted_iota(jnp.int32, (n_tok, NSLOT), 1).astype(
                jnp.float32
            )
        ).astype(jnp.bfloat16)
        yf = yin_s[...].reshape(NSLOT, d_ff)
        out_ref[...] = jnp.dot(
            ptmat, yf, preferred_element_type=jnp.float32
        )

        for s in cnt_sends + x_sends + y_sends:
            s.wait_send()

    return pl.pallas_call(
        body,
        out_shape=jax.ShapeDtypeStruct((n_tok, d_ff), jnp.float32),
        in_specs=[pl.BlockSpec(memory_space=pltpu.VMEM)] * 4,
        out_specs=pl.BlockSpec(memory_space=pltpu.VMEM),
        scratch_shapes=[
            pltpu.VMEM((N_DEV, CAP_G, D_BLK), jnp.bfloat16),
            pltpu.VMEM((N_DEV, CAP_G, D_BLK), jnp.bfloat16),
            pltpu.VMEM((N_DEV, CAP_G, d_ff), jnp.bfloat16),
            pltpu.VMEM((N_DEV, CAP_G, d_ff), jnp.bfloat16),
            pltpu.VMEM((1, 128), jnp.float32),
            pltpu.SemaphoreType.DMA((N_DEV,)),
            pltpu.SemaphoreType.DMA((N_DEV,)),
            pltpu.SemaphoreType.DMA((N_DEV,)),
            pltpu.SemaphoreType.DMA((N_DEV,)),
            pltpu.SemaphoreType.DMA((N_DEV,)),
            pltpu.SemaphoreType.DMA((N_DEV,)),
            pltpu.VMEM((N_DEV, 128), jnp.float32),
        ],
        compiler_params=pltpu.CompilerParams(collective_id=0),
    )(xb, wb, route_col, route_row)


# device time: 57782 ns/iter; 2.2362x vs baseline; 1.0413x over previous
import jax
import jax.numpy as jnp
from jax import lax
from jax.experimental import pallas as pl
from jax.experimental.pallas import tpu as pltpu

N_DEV = 4
N_EXP = 16
EXP_PER_DEV = 4
CAPACITY = 204
CAP_G = 384
NSLOT = N_DEV * CAP_G
META_L = 128


def kernel(x, router_W, route_idx, expert_W):
    del router_W
    n_tok, d_model = x.shape
    _, _, d_ff = expert_W.shape

    xb = x.astype(jnp.bfloat16)
    wb = expert_W.astype(jnp.bfloat16)
    route_col = route_idx
    route_row = route_idx.reshape(1, n_tok)

    D_BLK = d_model + META_L

    def body(
        xb_ref, wb_ref, rc_ref, rr_ref, out_ref,
        xc_s, xin_s, yb_s, yin_s, cnt_s,
        cnt_send, cnt_recv, xa_send, xa_recv, ya_send, ya_recv,
        counts_all,
    ):
        my = lax.axis_index("i")

        barrier = pltpu.get_barrier_semaphore()
        for o in range(1, N_DEV):
            pl.semaphore_signal(
                barrier, inc=1,
                device_id=(jnp.mod(my + o, N_DEV),),
                device_id_type=pl.DeviceIdType.MESH,
            )
        pl.semaphore_wait(barrier, N_DEV - 1)

        rc = rc_ref[...]
        rr = rr_ref[...]

        oh128 = (
            rc == lax.broadcasted_iota(jnp.int32, (n_tok, 128), 1)
        ).astype(jnp.float32)
        cnt_s[...] = oh128.sum(axis=0, keepdims=True)

        cnt_sends = []
        for o in range(1, N_DEV):
            tgt = jnp.mod(my + o, N_DEV)
            s = pltpu.make_async_remote_copy(
                src_ref=cnt_s,
                dst_ref=counts_all.at[pl.ds(my, 1)],
                send_sem=cnt_send.at[o],
                recv_sem=cnt_recv.at[my],
                device_id=(tgt,),
                device_id_type=pl.DeviceIdType.MESH,
            )
            s.start()
            cnt_sends.append(s)

        lower = (
            lax.broadcasted_iota(jnp.int32, (n_tok, n_tok), 0)
            >= lax.broadcasted_iota(jnp.int32, (n_tok, n_tok), 1)
        ).astype(jnp.bfloat16)

        oh16 = (
            rc == lax.broadcasted_iota(jnp.int32, (n_tok, N_EXP), 1)
        )
        oh16b = oh16.astype(jnp.bfloat16)
        oh16f = oh16.astype(jnp.float32)
        csum16 = jnp.dot(lower, oh16b, preferred_element_type=jnp.float32)
        rank = ((csum16 - oh16f) * oh16f).sum(axis=1, keepdims=True)

        gc = rc // EXP_PER_DEV
        ohg_c = (
            gc == lax.broadcasted_iota(jnp.int32, (n_tok, N_DEV), 1)
        )
        ohg_cb = ohg_c.astype(jnp.bfloat16)
        ohg_cf = ohg_c.astype(jnp.float32)
        csg_c = jnp.dot(lower, ohg_cb, preferred_element_type=jnp.float32)
        rkg_c = ((csg_c - ohg_cf) * ohg_cf).sum(axis=1, keepdims=True)
        slot_col = jnp.where(
            rkg_c < CAP_G,
            gc.astype(jnp.float32) * CAP_G + rkg_c,
            float(NSLOT),
        )

        upper = (
            lax.broadcasted_iota(jnp.int32, (n_tok, n_tok), 0)
            <= lax.broadcasted_iota(jnp.int32, (n_tok, n_tok), 1)
        ).astype(jnp.bfloat16)
        gr = rr // EXP_PER_DEV
        ohg_r = (
            gr == lax.broadcasted_iota(jnp.int32, (N_DEV, n_tok), 0)
        )
        ohg_rb = ohg_r.astype(jnp.bfloat16)
        ohg_rf = ohg_r.astype(jnp.float32)
        csg_r = jnp.dot(ohg_rb, upper, preferred_element_type=jnp.float32)
        rkg_r = ((csg_r - ohg_rf) * ohg_rf).sum(axis=0, keepdims=True)
        slot_row = jnp.where(
            rkg_r < CAP_G,
            gr.astype(jnp.float32) * CAP_G + rkg_r,
            float(NSLOT),
        )

        rhi = jnp.floor(rank / 256.0)
        rlo = rank - 256.0 * rhi
        lane = lax.broadcasted_iota(jnp.int32, (1, META_L), 1)
        meta = (
            rc.astype(jnp.float32) * (lane == 0)
            + rhi * (lane == 1)
            + rlo * (lane == 2)
        ).astype(jnp.bfloat16)

        pmat = (
            lax.broadcasted_iota(jnp.int32, (NSLOT, n_tok), 0).astype(
                jnp.float32
            )
            == slot_row
        ).astype(jnp.bfloat16)
        xmeta = jnp.concatenate([xb_ref[...], meta], axis=1)
        xc_s[...] = (
            jnp.dot(pmat, xmeta, preferred_element_type=jnp.float32)
            .astype(jnp.bfloat16)
            .reshape(N_DEV, CAP_G, D_BLK)
        )

        xin_s[pl.ds(my, 1)] = xc_s[pl.ds(my, 1)]
        x_sends = []
        for o in range(1, N_DEV):
            tgt = jnp.mod(my + o, N_DEV)
            sx = pltpu.make_async_remote_copy(
                src_ref=xc_s.at[pl.ds(tgt, 1)],
                dst_ref=xin_s.at[pl.ds(my, 1)],
                send_sem=xa_send.at[o],
                recv_sem=xa_recv.at[my],
                device_id=(tgt,),
                device_id_type=pl.DeviceIdType.MESH,
            )
            sx.start()
            x_sends.append(sx)

        counts_all[pl.ds(my, 1)] = cnt_s[...]
        for o in range(1, N_DEV):
            src_dev = jnp.mod(my + o, N_DEV)
            r = pltpu.make_async_remote_copy(
                src_ref=cnt_s,
                dst_ref=counts_all.at[pl.ds(src_dev, 1)],
                send_sem=cnt_send.at[0],
                recv_sem=cnt_recv.at[src_dev],
                device_id=(src_dev,),
                device_id_type=pl.DeviceIdType.MESH,
            )
            r.wait_recv()

        ca = counts_all[...]
        dev_iota = lax.broadcasted_iota(jnp.int32, (N_DEV, 128), 0)
        lane16 = lax.broadcasted_iota(jnp.int32, (1, N_EXP), 1).astype(
            jnp.float32
        )

        wflat = wb_ref[...].reshape(EXP_PER_DEV * d_model, d_ff)

        def compute_block(s, dst_ref):
            blk = xin_s[pl.ds(s, 1)].reshape(CAP_G, D_BLK)
            xv = blk[:, :d_model]
            ev = blk[:, d_model:d_model + 1].astype(jnp.float32)
            rk = (
                blk[:, d_model + 1:d_model + 2].astype(jnp.float32) * 256.0
                + blk[:, d_model + 2:d_model + 3].astype(jnp.float32)
            )
            offs_vec = jnp.where(dev_iota < s, ca, 0.0).sum(
                axis=0, keepdims=True
            )[:, :N_EXP]
            ohs = (ev == lane16).astype(jnp.float32)
            offs_row = (ohs * offs_vec).sum(axis=1, keepdims=True)
            keep = (rk + offs_row) < float(CAPACITY)
            xms = []
            for k in range(EXP_PER_DEV):
                e_f = (my * EXP_PER_DEV + k).astype(jnp.float32)
                m = (ev == e_f) & keep
                xms.append(xv * m.astype(jnp.bfloat16))
            x4 = jnp.concatenate(xms, axis=1)
            acc = jnp.dot(x4, wflat, preferred_element_type=jnp.float32)
            dst_ref[pl.ds(s, 1)] = acc.astype(jnp.bfloat16)[None]

        compute_block(my, yin_s)

        y_sends = []
        for o in (1, 3, 2):
            s = jnp.mod(my + o, N_DEV)
            rx = pltpu.make_async_remote_copy(
                src_ref=xc_s.at[pl.ds(s, 1)],
                dst_ref=xin_s.at[pl.ds(s, 1)],
                send_sem=xa_send.at[0],
                recv_sem=xa_recv.at[s],
                device_id=(s,),
                device_id_type=pl.DeviceIdType.MESH,
            )
            rx.wait_recv()
            compute_block(s, yb_s)
            sy = pltpu.make_async_remote_copy(
                src_ref=yb_s.at[pl.ds(s, 1)],
                dst_ref=yin_s.at[pl.ds(my, 1)],
                send_sem=ya_send.at[o],
                recv_sem=ya_recv.at[my],
                device_id=(s,),
                device_id_type=pl.DeviceIdType.MESH,
            )
            sy.start()
            y_sends.append(sy)

        ptmat = (
            slot_col
            == lax.broadcasted_iota(jnp.int32, (n_tok, NSLOT), 1).astype(
                jnp.float32
            )
        ).astype(jnp.bfloat16)

        for o in (1, 3, 2):
            g = jnp.mod(my + o, N_DEV)
            ry = pltpu.make_async_remote_copy(
                src_ref=yb_s.at[pl.ds(g, 1)],
                dst_ref=yin_s.at[pl.ds(g, 1)],
                send_sem=ya_send.at[0],
                recv_sem=ya_recv.at[g],
                device_id=(g,),
                device_id_type=pl.DeviceIdType.MESH,
            )
            ry.wait_recv()

        yf = yin_s[...].reshape(NSLOT, d_ff)
        out_ref[...] = jnp.dot(
            ptmat, yf, preferred_element_type=jnp.float32
        )

        for s in cnt_sends + x_sends + y_sends:
            s.wait_send()

    return pl.pallas_call(
        body,
        out_shape=jax.ShapeDtypeStruct((n_tok, d_ff), jnp.float32),
        in_specs=[pl.BlockSpec(memory_space=pltpu.VMEM)] * 4,
        out_specs=pl.BlockSpec(memory_space=pltpu.VMEM),
        scratch_shapes=[
            pltpu.VMEM((N_DEV, CAP_G, D_BLK), jnp.bfloat16),
            pltpu.VMEM((N_DEV, CAP_G, D_BLK), jnp.bfloat16),
            pltpu.VMEM((N_DEV, CAP_G, d_ff), jnp.bfloat16),
            pltpu.VMEM((N_DEV, CAP_G, d_ff), jnp.bfloat16),
            pltpu.VMEM((1, 128), jnp.float32),
            pltpu.SemaphoreType.DMA((N_DEV,)),
            pltpu.SemaphoreType.DMA((N_DEV,)),
            pltpu.SemaphoreType.DMA((N_DEV,)),
            pltpu.SemaphoreType.DMA((N_DEV,)),
            pltpu.SemaphoreType.DMA((N_DEV,)),
            pltpu.SemaphoreType.DMA((N_DEV,)),
            pltpu.VMEM((N_DEV, 128), jnp.float32),
        ],
        compiler_params=pltpu.CompilerParams(collective_id=0),
    )(xb, wb, route_col, route_row)
